# Initial kernel scaffold; baseline (speedup 1.0000x reference)
import functools

import jax
import jax.numpy as jnp
import numpy as np
from jax import lax
from jax.experimental import pallas as pl
from jax.experimental.pallas import tpu as pltpu
from jax.experimental.pallas import tpu_sc as plsc

N = 10000
E = 160000
LIN = 256
OUT = 256
HEADS = 4
HO = HEADS * OUT
HID = 128
NPAD = 10240
NB = 512
GRID = NPAD // NB
NSUB = 32
DPS = NPAD // NSUB
RPW = DPS * (NSUB - 1) + 328
F32 = jnp.float32


def _leaky(x):
    return jnp.where(x > 0, x, 0.01 * x)


def _dot(a, b):
    return jnp.dot(a, b, preferred_element_type=F32)


def _enc_body(prop_r, cat_r, tw_r, de_r, wn, bn, wb, bb, wt, bt, wd, bd,
              w1, b1, x_r):
    un = _leaky(_dot(prop_r[...], wn[...]) + bn[...])
    ub = _leaky(_dot(cat_r[...], wb[...]) + bb[...])
    ut = _leaky(_dot(tw_r[...], wt[...]) + bt[...])
    ud = _leaky(_dot(de_r[...], wd[...]) + bd[...])
    u = jnp.concatenate([un, ub, ut, ud], axis=1)
    x_r[...] = _leaky(_dot(u, w1[...]) + b1[...])


def _row_spec(c):
    return pl.BlockSpec((NB, c), lambda i: (i, 0))


def _full_spec(shape):
    return pl.BlockSpec(shape, lambda i: tuple(0 for _ in shape))


def _encode(prop, cat, tw, de, p):
    wn, bn = p["in_num"]["W"], p["in_num"]["b"].reshape(1, -1)
    wb, bb = p["in_bool"]["W"], p["in_bool"]["b"].reshape(1, -1)
    wt, bt = p["in_tweet"]["W"], p["in_tweet"]["b"].reshape(1, -1)
    wd, bd = p["in_des"]["W"], p["in_des"]["b"].reshape(1, -1)
    w1, b1 = p["linear1"]["W"], p["linear1"]["b"].reshape(1, -1)
    ins = [prop, cat, tw, de, wn, bn, wb, bb, wt, bt, wd, bd, w1, b1]
    in_specs = [_row_spec(5), _row_spec(3), _row_spec(768), _row_spec(768)]
    in_specs += [_full_spec(a.shape) for a in ins[4:]]
    return pl.pallas_call(
        _enc_body, grid=(GRID,),
        in_specs=in_specs,
        out_specs=_row_spec(LIN),
        out_shape=jax.ShapeDtypeStruct((NPAD, LIN), F32),
    )(*ins)


def _qkvs_body(from_sems, x_or_sem0, *rest):
    if from_sems:
        sem0_r, sem1_r, coef_r = x_or_sem0, rest[0], rest[1]
        w = rest[2:18]
        outs = rest[18:]
        x2_r = outs[0]
        outs = outs[1:]
        c0 = coef_r[0:1, 0:1]
        c1 = coef_r[0:1, 1:2]
        xv = _leaky(sem0_r[...] * c0 + sem1_r[...] * c1)
        x2_r[...] = xv
    else:
        w = rest[:16]
        outs = rest[16:]
        xv = x_or_sem0[...]
    for t in range(2):
        wq, bq, wk, bk, wv, bv, ws, bs = w[8 * t:8 * t + 8]
        q_r, k_r, v_r, s_r = outs[4 * t:4 * t + 4]
        q_r[...] = _dot(xv, wq[...]) + bq[...]
        k_r[...] = _dot(xv, wk[...]) + bk[...]
        v_r[...] = _dot(xv, wv[...]) + bv[...]
        s_r[...] = _dot(xv, ws[...]) + bs[...]


def _qkvs_weights(rp):
    w = []
    for t in range(2):
        tp = rp["trans"][t]
        for name in ("q", "k", "v", "skip"):
            w.append(tp[name]["W"])
            w.append(tp[name]["b"].reshape(1, -1))
    return w


def _qkvs(x, rgt_params):
    w = _qkvs_weights(rgt_params)
    ins = [x] + w
    in_specs = [_row_spec(LIN)] + [_full_spec(a.shape) for a in w]
    out_shapes, out_specs = [], []
    for t in range(2):
        for c in (HO, HO, HO, OUT):
            out_shapes.append(jax.ShapeDtypeStruct((NPAD, c), F32))
            out_specs.append(_row_spec(c))
    return pl.pallas_call(
        functools.partial(_qkvs_body, False), grid=(GRID,),
        in_specs=in_specs, out_specs=out_specs, out_shape=out_shapes,
    )(*ins)


def _qkvs_from_sems(sem0, sem1, coef, rgt_params):
    w = _qkvs_weights(rgt_params)
    ins = [sem0, sem1, coef] + w
    in_specs = [_row_spec(LIN), _row_spec(LIN), _full_spec((8, 128))]
    in_specs += [_full_spec(a.shape) for a in w]
    out_shapes = [jax.ShapeDtypeStruct((NPAD, LIN), F32)]
    out_specs = [_row_spec(LIN)]
    for t in range(2):
        for c in (HO, HO, HO, OUT):
            out_shapes.append(jax.ShapeDtypeStruct((NPAD, c), F32))
            out_specs.append(_row_spec(c))
    return pl.pallas_call(
        functools.partial(_qkvs_body, True), grid=(GRID,),
        in_specs=in_specs, out_specs=out_specs, out_shape=out_shapes,
    )(*ins)


def _gate_body(x_r, c0_r, c1_r, s0_r, s1_r, wgu, wgx, bg,
               l1w0, l1b0, l2r0, l1w1, l1b1, l2r1,
               sem0_r, sem1_r, coef_r, wacc):
    i = pl.program_id(0)
    xv = x_r[...]
    sems = []
    for t in range(2):
        u = (c0_r, c1_r)[t][...] + (s0_r, s1_r)[t][...]
        a = jax.nn.sigmoid(_dot(u, wgu[...]) + _dot(xv, wgx[...]) + bg[...])
        sems.append(jnp.tanh(u) * a + xv * (1.0 - a))
    sem0_r[...] = sems[0]
    sem1_r[...] = sems[1]
    rows = i * NB + lax.broadcasted_iota(jnp.int32, (NB, 1), 0)
    rmask = rows < N
    rio = lax.broadcasted_iota(jnp.int32, (8, 128), 0)
    cio = lax.broadcasted_iota(jnp.int32, (8, 128), 1)
    wblk = jnp.zeros((8, 128), F32)
    for p_i, (l1w, l1b, l2r) in enumerate(((l1w0, l1b0, l2r0),
                                           (l1w1, l1b1, l2r1))):
        for t in range(2):
            t1 = jnp.tanh(_dot(sems[t], l1w[...]) + l1b[...])
            hn = jnp.sum(t1 * l2r[...], axis=1, keepdims=True)
            tot = jnp.sum(jnp.where(rmask, hn, 0.0))
            wblk = wblk + jnp.where((rio == p_i) & (cio == t), tot, 0.0)

    @pl.when(i == 0)
    def _():
        wacc[...] = wblk

    @pl.when(i > 0)
    def _():
        wacc[...] = wacc[...] + wblk

    @pl.when(i == GRID - 1)
    def _():
        w = wacc[...] / float(N)
        e = jnp.exp(w)
        den = jnp.sum(jnp.where(cio < 2, e, 0.0), axis=1, keepdims=True)
        r = jnp.where(cio < 2, e / den, 0.0)
        csum = jnp.sum(jnp.where(rio < 2, r, 0.0), axis=0, keepdims=True)
        coef_r[...] = jnp.broadcast_to(csum * 0.5, (8, 128))


def _gate(x, conv0, conv1, skip0, skip1, rgt_params):
    gw = rgt_params["gate"]["W"]
    wgu, wgx = gw[:LIN], gw[LIN:]
    bg = rgt_params["gate"]["b"].reshape(1, -1)
    sem = rgt_params["sem"]
    l1w0, l1b0 = sem[0]["l1"]["W"], sem[0]["l1"]["b"].reshape(1, -1)
    l2r0 = sem[0]["l2"]["W"].T
    l1w1, l1b1 = sem[1]["l1"]["W"], sem[1]["l1"]["b"].reshape(1, -1)
    l2r1 = sem[1]["l2"]["W"].T
    ins = [x, conv0, conv1, skip0, skip1, wgu, wgx, bg,
           l1w0, l1b0, l2r0, l1w1, l1b1, l2r1]
    in_specs = [_row_spec(LIN)] * 5 + [_full_spec(a.shape) for a in ins[5:]]
    out_shapes = [jax.ShapeDtypeStruct((NPAD, LIN), F32),
                  jax.ShapeDtypeStruct((NPAD, LIN), F32),
                  jax.ShapeDtypeStruct((8, 128), F32)]
    out_specs = [_row_spec(LIN), _row_spec(LIN), _full_spec((8, 128))]
    return pl.pallas_call(
        _gate_body, grid=(GRID,),
        in_specs=in_specs, out_specs=out_specs, out_shape=out_shapes,
        scratch_shapes=[pltpu.VMEM((8, 128), F32)],
    )(*ins)


def _final_body(sem0_r, sem1_r, coef_r, wo1, bo1, wo2, bo2, y_r):
    c0 = coef_r[0:1, 0:1]
    c1 = coef_r[0:1, 1:2]
    xv = _leaky(sem0_r[...] * c0 + sem1_r[...] * c1)
    h = _leaky(_dot(xv, wo1[...]) + bo1[...])
    y_r[...] = _dot(h, wo2[...]) + bo2[...]


def _final(sem0, sem1, coef, p):
    wo1, bo1 = p["out1"]["W"], p["out1"]["b"].reshape(1, -1)
    wo2 = jnp.pad(p["out2"]["W"], ((0, 0), (0, 126)))
    bo2 = jnp.pad(p["out2"]["b"].reshape(1, -1), ((0, 0), (0, 126)))
    ins = [sem0, sem1, coef, wo1, bo1, wo2, bo2]
    in_specs = [_row_spec(LIN), _row_spec(LIN), _full_spec((8, 128))]
    in_specs += [_full_spec(a.shape) for a in ins[3:]]
    return pl.pallas_call(
        _final_body, grid=(GRID,),
        in_specs=in_specs,
        out_specs=_row_spec(128),
        out_shape=jax.ShapeDtypeStruct((NPAD, 128), F32),
    )(*ins)


def _conv_body(rp_hbm, src_hbm, q0, k0, v0, q1, k1, v1, out_hbm,
               rp_v, win, idx, qbuf, kbuf, vbuf, acc, orow,
               semk, semv):
    wid = lax.axis_index("s") * 2 + lax.axis_index("c")
    d0 = wid * DPS
    ndst = jnp.minimum(DPS, N - d0)
    lanes = lax.iota(jnp.int32, 16)
    minf = jnp.full((16,), -jnp.inf, F32)
    zero16 = jnp.zeros((16,), F32)
    for t in range(2):
        pltpu.sync_copy(rp_hbm.at[t, pl.ds(d0, 328)], rp_v.at[t])
    for t, (qh, kh, vh) in enumerate(((q0, k0, v0), (q1, k1, v1))):

        def grp_body(jg, _, t=t, qh=qh, kh=kh, vh=vh):
            dg = d0 + 16 * jg
            pltpu.sync_copy(qh.at[pl.ds(dg, 16)], qbuf)
            nloc = jnp.minimum(16, ndst - 16 * jg)

            def node_body(jl, _2):
                j = 16 * jg + jl
                d = d0 + j
                start = rp_v[t, j]
                end = rp_v[t, j + 1]
                cnt = end - start
                for g in range(64):
                    acc[pl.ds(16 * g, 16)] = zero16
                nch = lax.div(cnt + 15, 16)

                def chunk_body(c, carry):
                    s0_ = start + 16 * c
                    valid = jnp.minimum(16, cnt - 16 * c)
                    a = lax.div(s0_, 8) * 8
                    off = s0_ - a
                    pltpu.sync_copy(src_hbm.at[pl.ds(a, 24)], win)
                    iv = plsc.load_gather(win, [lanes + off])
                    iv = jnp.where(lanes < valid, iv, 0)
                    idx[...] = iv
                    ck = pltpu.async_copy(kh.at[idx], kbuf, semk)
                    cv = pltpu.async_copy(vh.at[idx], vbuf, semv)
                    ck.wait()
                    cv.wait()
                    newm, news, ps = [], [], []
                    for h in range(4):
                        qv = [qbuf[jl, pl.ds(256 * h + 16 * g, 16)]
                              for g in range(16)]

                        def ebody(jj, sc, h=h, qv=qv):
                            dot = qv[0] * kbuf[jj, pl.ds(256 * h, 16)]
                            for g in range(1, 16):
                                dot = dot + qv[g] * kbuf[
                                    jj, pl.ds(256 * h + 16 * g, 16)]
                            ssc = jnp.sum(dot) * (1.0 / 16.0)
                            return jnp.where(lanes == jj,
                                             jnp.full((16,), ssc, F32), sc)

                        sch = lax.fori_loop(0, valid, ebody, minf)
                        m_h = carry[h]
                        s_h = carry[4 + h]
                        mn = jnp.maximum(m_h,
                                         jnp.full((16,), jnp.max(sch), F32))
                        scale = jnp.exp(m_h - mn)
                        p = jnp.exp(sch - mn)
                        newm.append(mn)
                        news.append(s_h * scale +
                                    jnp.full((16,), jnp.sum(p), F32))
                        ps.append(p)
                        for g in range(16):
                            sl = pl.ds(256 * h + 16 * g, 16)
                            acc[sl] = acc[sl] * scale

                    def abody(jj, _3):
                        for h in range(4):
                            pj = jnp.sum(jnp.where(lanes == jj, ps[h], 0.0))
                            pv = jnp.full((16,), pj, F32)
                            for g in range(16):
                                sl = pl.ds(256 * h + 16 * g, 16)
                                plsc.addupdate(acc.at[sl],
                                               pv * vbuf[jj, sl])
                        return 0

                    lax.fori_loop(0, valid, abody, 0)
                    return tuple(newm + news)

                fin = lax.fori_loop(0, nch, chunk_body,
                                    (minf,) * 4 + (zero16,) * 4)
                invs = [1.0 / (fin[4 + h] + 1e-16) for h in range(4)]
                for g in range(16):
                    o = zero16
                    for h in range(4):
                        o = o + acc[pl.ds(256 * h + 16 * g, 16)] * invs[h]
                    orow[pl.ds(16 * g, 16)] = o * 0.25
                pltpu.sync_copy(orow, out_hbm.at[t, d])
                return 0

            lax.fori_loop(0, nloc, node_body, 0)
            return 0

        lax.fori_loop(0, lax.div(ndst + 15, 16), grp_body, 0)


def _sc_conv(rp2, ssrcp, q0, k0, v0, q1, k1, v1):
    mesh = plsc.VectorSubcoreMesh(core_axis_name="c", subcore_axis_name="s")
    kfn = pl.kernel(
        _conv_body,
        out_type=jax.ShapeDtypeStruct((2, NPAD, OUT), F32),
        mesh=mesh,
        scratch_types=[
            pltpu.VMEM((2, 328), jnp.int32),
            pltpu.VMEM((24,), jnp.int32),
            pltpu.VMEM((16,), jnp.int32),
            pltpu.VMEM((16, HO), F32),
            pltpu.VMEM((16, HO), F32),
            pltpu.VMEM((16, HO), F32),
            pltpu.VMEM((HO,), F32),
            pltpu.VMEM((OUT,), F32),
            pltpu.SemaphoreType.DMA,
            pltpu.SemaphoreType.DMA,
        ],
    )
    return kfn(rp2, ssrcp, q0, k0, v0, q1, k1, v1)


def kernel(cat_features, prop_features, tweet_features, des_features,
           edge_index, edge_type, params):
    src = edge_index[0].astype(jnp.int32)
    dst = edge_index[1].astype(jnp.int32)
    et = edge_type.astype(jnp.int32)
    key = et * N + dst
    order = jnp.argsort(key)
    skey = jnp.take(key, order)
    ssrc = jnp.take(src, order)
    rp = jnp.searchsorted(skey, jnp.arange(2 * N + 1, dtype=jnp.int32))
    rp = rp.astype(jnp.int32)
    rp2 = jnp.stack([rp[:N + 1], rp[N:2 * N + 1]])
    rp2 = jnp.pad(rp2, ((0, 0), (0, RPW - (N + 1))), mode="edge")
    ssrcp = jnp.pad(ssrc, (0, 24))

    x = _encode(prop_features, cat_features, tweet_features, des_features,
                params)
    sem0 = sem1 = coef = None
    for li, rgt in enumerate((params["rgt1"], params["rgt2"])):
        if li == 0:
            q0, k0, v0, s0, q1, k1, v1, s1 = _qkvs(x, rgt)
        else:
            x, q0, k0, v0, s0, q1, k1, v1, s1 = _qkvs_from_sems(
                sem0, sem1, coef, rgt)
        conv = _sc_conv(rp2, ssrcp, q0, k0, v0, q1, k1, v1)
        sem0, sem1, coef = _gate(x, conv[0], conv[1], s0, s1, rgt)
    y = _final(sem0, sem1, coef, params)
    return y[:N, :2]


# speedup vs baseline: 5.4723x; 5.4723x over previous
import functools

import jax
import jax.numpy as jnp
import numpy as np
from jax import lax
from jax.experimental import pallas as pl
from jax.experimental.pallas import tpu as pltpu
from jax.experimental.pallas import tpu_sc as plsc

N = 10000
E = 160000
LIN = 256
OUT = 256
HEADS = 4
HO = HEADS * OUT
HID = 128
NPAD = 10240
NB = 512
GRID = NPAD // NB
NSUB = 32
DPS = NPAD // NSUB
RPW = DPS * (NSUB - 1) + 336
F32 = jnp.float32


def _leaky(x):
    return jnp.where(x > 0, x, 0.01 * x)


def _dot(a, b):
    return jnp.dot(a, b, preferred_element_type=F32)


def _enc_body(prop_r, cat_r, tw_r, de_r, wn, bn, wb, bb, wt, bt, wd, bd,
              w1, b1, x_r):
    un = _leaky(_dot(prop_r[...], wn[...]) + bn[...])
    ub = _leaky(_dot(cat_r[...], wb[...]) + bb[...])
    ut = _leaky(_dot(tw_r[...], wt[...]) + bt[...])
    ud = _leaky(_dot(de_r[...], wd[...]) + bd[...])
    u = jnp.concatenate([un, ub, ut, ud], axis=1)
    x_r[...] = _leaky(_dot(u, w1[...]) + b1[...])


def _row_spec(c):
    return pl.BlockSpec((NB, c), lambda i: (i, 0))


def _full_spec(shape):
    return pl.BlockSpec(shape, lambda i: tuple(0 for _ in shape))


def _encode(prop, cat, tw, de, p):
    wn, bn = p["in_num"]["W"], p["in_num"]["b"].reshape(1, -1)
    wb, bb = p["in_bool"]["W"], p["in_bool"]["b"].reshape(1, -1)
    wt, bt = p["in_tweet"]["W"], p["in_tweet"]["b"].reshape(1, -1)
    wd, bd = p["in_des"]["W"], p["in_des"]["b"].reshape(1, -1)
    w1, b1 = p["linear1"]["W"], p["linear1"]["b"].reshape(1, -1)
    ins = [prop, cat, tw, de, wn, bn, wb, bb, wt, bt, wd, bd, w1, b1]
    in_specs = [_row_spec(5), _row_spec(3), _row_spec(768), _row_spec(768)]
    in_specs += [_full_spec(a.shape) for a in ins[4:]]
    return pl.pallas_call(
        _enc_body, grid=(GRID,),
        in_specs=in_specs,
        out_specs=_row_spec(LIN),
        out_shape=jax.ShapeDtypeStruct((NPAD, LIN), F32),
    )(*ins)


def _qkvs_body(from_sems, x_or_sem0, *rest):
    if from_sems:
        sem0_r, sem1_r, coef_r = x_or_sem0, rest[0], rest[1]
        w = rest[2:18]
        outs = rest[18:]
        x2_r = outs[0]
        outs = outs[1:]
        c0 = coef_r[0:1, 0:1]
        c1 = coef_r[0:1, 1:2]
        xv = _leaky(sem0_r[...] * c0 + sem1_r[...] * c1)
        x2_r[...] = xv
    else:
        w = rest[:16]
        outs = rest[16:]
        xv = x_or_sem0[...]
    for t in range(2):
        wq, bq, wk, bk, wv, bv, ws, bs = w[8 * t:8 * t + 8]
        q_r, k_r, v_r, s_r = outs[4 * t:4 * t + 4]
        q_r[...] = _dot(xv, wq[...]) + bq[...]
        k_r[...] = _dot(xv, wk[...]) + bk[...]
        v_r[...] = _dot(xv, wv[...]) + bv[...]
        s_r[...] = _dot(xv, ws[...]) + bs[...]


def _qkvs_weights(rp):
    w = []
    for t in range(2):
        tp = rp["trans"][t]
        for name in ("q", "k", "v", "skip"):
            w.append(tp[name]["W"])
            w.append(tp[name]["b"].reshape(1, -1))
    return w


def _qkvs(x, rgt_params):
    w = _qkvs_weights(rgt_params)
    ins = [x] + w
    in_specs = [_row_spec(LIN)] + [_full_spec(a.shape) for a in w]
    out_shapes, out_specs = [], []
    for t in range(2):
        for c in (HO, HO, HO, OUT):
            out_shapes.append(jax.ShapeDtypeStruct((NPAD, c), F32))
            out_specs.append(_row_spec(c))
    return pl.pallas_call(
        functools.partial(_qkvs_body, False), grid=(GRID,),
        in_specs=in_specs, out_specs=out_specs, out_shape=out_shapes,
    )(*ins)


def _qkvs_from_sems(sem0, sem1, coef, rgt_params):
    w = _qkvs_weights(rgt_params)
    ins = [sem0, sem1, coef] + w
    in_specs = [_row_spec(LIN), _row_spec(LIN), _full_spec((8, 128))]
    in_specs += [_full_spec(a.shape) for a in w]
    out_shapes = [jax.ShapeDtypeStruct((NPAD, LIN), F32)]
    out_specs = [_row_spec(LIN)]
    for t in range(2):
        for c in (HO, HO, HO, OUT):
            out_shapes.append(jax.ShapeDtypeStruct((NPAD, c), F32))
            out_specs.append(_row_spec(c))
    return pl.pallas_call(
        functools.partial(_qkvs_body, True), grid=(GRID,),
        in_specs=in_specs, out_specs=out_specs, out_shape=out_shapes,
    )(*ins)


def _gate_body(x_r, c0_r, c1_r, s0_r, s1_r, wgu, wgx, bg,
               l1w0, l1b0, l2r0, l1w1, l1b1, l2r1,
               sem0_r, sem1_r, coef_r, wacc):
    i = pl.program_id(0)
    xv = x_r[...]
    sems = []
    for t in range(2):
        u = (c0_r, c1_r)[t][...] + (s0_r, s1_r)[t][...]
        a = jax.nn.sigmoid(_dot(u, wgu[...]) + _dot(xv, wgx[...]) + bg[...])
        sems.append(jnp.tanh(u) * a + xv * (1.0 - a))
    sem0_r[...] = sems[0]
    sem1_r[...] = sems[1]
    rows = i * NB + lax.broadcasted_iota(jnp.int32, (NB, 1), 0)
    rmask = rows < N
    rio = lax.broadcasted_iota(jnp.int32, (8, 128), 0)
    cio = lax.broadcasted_iota(jnp.int32, (8, 128), 1)
    wblk = jnp.zeros((8, 128), F32)
    for p_i, (l1w, l1b, l2r) in enumerate(((l1w0, l1b0, l2r0),
                                           (l1w1, l1b1, l2r1))):
        for t in range(2):
            t1 = jnp.tanh(_dot(sems[t], l1w[...]) + l1b[...])
            hn = jnp.sum(t1 * l2r[...], axis=1, keepdims=True)
            tot = jnp.sum(jnp.where(rmask, hn, 0.0))
            wblk = wblk + jnp.where((rio == p_i) & (cio == t), tot, 0.0)

    @pl.when(i == 0)
    def _():
        wacc[...] = wblk

    @pl.when(i > 0)
    def _():
        wacc[...] = wacc[...] + wblk

    @pl.when(i == GRID - 1)
    def _():
        w = wacc[...] / float(N)
        e = jnp.exp(w)
        den = jnp.sum(jnp.where(cio < 2, e, 0.0), axis=1, keepdims=True)
        r = jnp.where(cio < 2, e / den, 0.0)
        csum = jnp.sum(jnp.where(rio < 2, r, 0.0), axis=0, keepdims=True)
        coef_r[...] = jnp.broadcast_to(csum * 0.5, (8, 128))


def _gate(x, conv0, conv1, skip0, skip1, rgt_params):
    gw = rgt_params["gate"]["W"]
    wgu, wgx = gw[:LIN], gw[LIN:]
    bg = rgt_params["gate"]["b"].reshape(1, -1)
    sem = rgt_params["sem"]
    l1w0, l1b0 = sem[0]["l1"]["W"], sem[0]["l1"]["b"].reshape(1, -1)
    l2r0 = sem[0]["l2"]["W"].T
    l1w1, l1b1 = sem[1]["l1"]["W"], sem[1]["l1"]["b"].reshape(1, -1)
    l2r1 = sem[1]["l2"]["W"].T
    ins = [x, conv0, conv1, skip0, skip1, wgu, wgx, bg,
           l1w0, l1b0, l2r0, l1w1, l1b1, l2r1]
    in_specs = [_row_spec(LIN)] * 5 + [_full_spec(a.shape) for a in ins[5:]]
    out_shapes = [jax.ShapeDtypeStruct((NPAD, LIN), F32),
                  jax.ShapeDtypeStruct((NPAD, LIN), F32),
                  jax.ShapeDtypeStruct((8, 128), F32)]
    out_specs = [_row_spec(LIN), _row_spec(LIN), _full_spec((8, 128))]
    return pl.pallas_call(
        _gate_body, grid=(GRID,),
        in_specs=in_specs, out_specs=out_specs, out_shape=out_shapes,
        scratch_shapes=[pltpu.VMEM((8, 128), F32)],
    )(*ins)


def _final_body(sem0_r, sem1_r, coef_r, wo1, bo1, wo2, bo2, y_r):
    c0 = coef_r[0:1, 0:1]
    c1 = coef_r[0:1, 1:2]
    xv = _leaky(sem0_r[...] * c0 + sem1_r[...] * c1)
    h = _leaky(_dot(xv, wo1[...]) + bo1[...])
    y_r[...] = _dot(h, wo2[...]) + bo2[...]


def _final(sem0, sem1, coef, p):
    wo1, bo1 = p["out1"]["W"], p["out1"]["b"].reshape(1, -1)
    wo2 = jnp.pad(p["out2"]["W"], ((0, 0), (0, 126)))
    bo2 = jnp.pad(p["out2"]["b"].reshape(1, -1), ((0, 0), (0, 126)))
    ins = [sem0, sem1, coef, wo1, bo1, wo2, bo2]
    in_specs = [_row_spec(LIN), _row_spec(LIN), _full_spec((8, 128))]
    in_specs += [_full_spec(a.shape) for a in ins[3:]]
    return pl.pallas_call(
        _final_body, grid=(GRID,),
        in_specs=in_specs,
        out_specs=_row_spec(128),
        out_shape=jax.ShapeDtypeStruct((NPAD, 128), F32),
    )(*ins)


def _conv_body(rp_hbm, src_hbm, q0, k0, v0, q1, k1, v1, out_hbm,
               rp_v, win, idx, qbuf, kbuf, vbuf, acc, orow, sbuf, svec,
               semk, semv):
    wid = lax.axis_index("s") * 2 + lax.axis_index("c")
    d0 = wid * DPS
    ndst = jnp.minimum(DPS, N - d0)
    lanes = lax.iota(jnp.int32, 16)
    zero16 = jnp.zeros((16,), F32)
    minf = jnp.float32(-jnp.inf)
    sbuf[pl.ds(16, 16)] = zero16
    for t in range(2):
        pltpu.sync_copy(rp_hbm.at[pl.ds(t * RPW + d0, 336)],
                        rp_v.at[pl.ds(336 * t, 336)])

    def _lane_sum(v):
        for k in (8, 4, 2, 1):
            sbuf[pl.ds(0, 16)] = v
            v = v + sbuf[pl.ds(k, 16)]
        return v[0]

    for t, (qh, kh, vh) in enumerate(((q0, k0, v0), (q1, k1, v1))):

        def grp_body(jg, _, t=t, qh=qh, kh=kh, vh=vh):
            dg = d0 + 16 * jg
            pltpu.sync_copy(qh.at[pl.ds(dg, 16)], qbuf)
            nloc = jnp.minimum(16, ndst - 16 * jg)

            def node_body(jl, _2):
                j = 16 * jg + jl
                d = d0 + j
                rpvec = rp_v[pl.ds(336 * t + j, 16)]
                start = rpvec[0]
                end = rpvec[1]
                cnt = end - start
                for g in range(64):
                    acc[pl.ds(16 * g, 16)] = zero16
                nch = lax.div(cnt + 15, 16)

                def chunk_body(c, carry):
                    s0_ = start + 16 * c
                    valid = jnp.minimum(16, cnt - 16 * c)
                    a = lax.div(s0_, 8) * 8
                    off = s0_ - a
                    pltpu.sync_copy(src_hbm.at[pl.ds(a, 24)], win)
                    iv = win[pl.ds(off, 16)]
                    iv = jnp.where(lanes < valid, iv, 0)
                    idx[...] = iv
                    ck = pltpu.async_copy(kh.at[idx], kbuf, semk)
                    cv = pltpu.async_copy(vh.at[idx], vbuf, semv)
                    ck.wait()
                    cv.wait()
                    newm, news = [], []
                    for h in range(4):
                        qv = [qbuf[jl, pl.ds(256 * h + 16 * g, 16)]
                              for g in range(16)]

                        def ebody(jj, m_h, h=h, qv=qv):
                            dot = qv[0] * kbuf[jj, pl.ds(256 * h, 16)]
                            for g in range(1, 16):
                                dot = dot + qv[g] * kbuf[
                                    jj, pl.ds(256 * h + 16 * g, 16)]
                            s = _lane_sum(dot) * (1.0 / 16.0)
                            svec[jj, pl.ds(16 * h, 16)] = jnp.full(
                                (16,), s, F32)
                            return jnp.maximum(m_h, s)

                        newm.append(lax.fori_loop(0, valid, ebody, carry[h]))
                    scales = []
                    for h in range(4):
                        scale = jnp.exp(jnp.full((16,), carry[h], F32)
                                        - jnp.full((16,), newm[h], F32))
                        scales.append(scale)
                        for g in range(16):
                            sl = pl.ds(256 * h + 16 * g, 16)
                            acc[sl] = acc[sl] * scale
                    for h in range(4):
                        mfull = jnp.full((16,), newm[h], F32)

                        def abody(jj, ps_h, h=h, mfull=mfull):
                            pv = jnp.exp(svec[jj, pl.ds(16 * h, 16)] - mfull)
                            for g in range(16):
                                sl = pl.ds(256 * h + 16 * g, 16)
                                plsc.addupdate(acc.at[sl],
                                               pv * vbuf[jj, sl])
                            return ps_h + pv

                        news.append(lax.fori_loop(
                            0, valid, abody,
                            carry[4 + h] * scales[h]))
                    return tuple(newm + news)

                fin = lax.fori_loop(0, nch, chunk_body,
                                    (minf,) * 4 + (zero16,) * 4)
                invs = [1.0 / (fin[4 + h] + 1e-16) for h in range(4)]
                for g in range(16):
                    o = zero16
                    for h in range(4):
                        o = o + acc[pl.ds(256 * h + 16 * g, 16)] * invs[h]
                    orow[pl.ds(16 * g, 16)] = o * 0.25
                pltpu.sync_copy(
                    orow, out_hbm.at[pl.ds((t * NPAD + d) * OUT, OUT)])
                return 0

            lax.fori_loop(0, nloc, node_body, 0)
            return 0

        lax.fori_loop(0, lax.div(ndst + 15, 16), grp_body, 0)


def _sc_conv(rp2, ssrcp, q0, k0, v0, q1, k1, v1):
    mesh = plsc.VectorSubcoreMesh(core_axis_name="c", subcore_axis_name="s")
    kfn = pl.kernel(
        _conv_body,
        out_type=jax.ShapeDtypeStruct((2 * NPAD * OUT,), F32),
        mesh=mesh,
        scratch_types=[
            pltpu.VMEM((672,), jnp.int32),
            pltpu.VMEM((24,), jnp.int32),
            pltpu.VMEM((16,), jnp.int32),
            pltpu.VMEM((16, HO), F32),
            pltpu.VMEM((16, HO), F32),
            pltpu.VMEM((16, HO), F32),
            pltpu.VMEM((HO,), F32),
            pltpu.VMEM((OUT,), F32),
            pltpu.VMEM((32,), F32),
            pltpu.VMEM((16, 64), F32),
            pltpu.SemaphoreType.DMA,
            pltpu.SemaphoreType.DMA,
        ],
    )
    out = kfn(rp2, ssrcp, q0, k0, v0, q1, k1, v1)
    return out.reshape(2, NPAD, OUT)


def kernel(cat_features, prop_features, tweet_features, des_features,
           edge_index, edge_type, params):
    src = edge_index[0].astype(jnp.int32)
    dst = edge_index[1].astype(jnp.int32)
    et = edge_type.astype(jnp.int32)
    key = et * N + dst
    order = jnp.argsort(key)
    skey = jnp.take(key, order)
    ssrc = jnp.take(src, order)
    rp = jnp.searchsorted(skey, jnp.arange(2 * N + 1, dtype=jnp.int32))
    rp = rp.astype(jnp.int32)
    rp2 = jnp.stack([rp[:N + 1], rp[N:2 * N + 1]])
    rp2 = jnp.pad(rp2, ((0, 0), (0, RPW - (N + 1))), mode="edge").reshape(-1)
    ssrcp = jnp.pad(ssrc, (0, 24))

    x = _encode(prop_features, cat_features, tweet_features, des_features,
                params)
    sem0 = sem1 = coef = None
    for li, rgt in enumerate((params["rgt1"], params["rgt2"])):
        if li == 0:
            q0, k0, v0, s0, q1, k1, v1, s1 = _qkvs(x, rgt)
        else:
            x, q0, k0, v0, s0, q1, k1, v1, s1 = _qkvs_from_sems(
                sem0, sem1, coef, rgt)
        conv = _sc_conv(rp2, ssrcp, q0, k0, v0, q1, k1, v1)
        sem0, sem1, coef = _gate(x, conv[0], conv[1], s0, s1, rgt)
    y = _final(sem0, sem1, coef, params)
    return y[:N, :2]


# speedup vs baseline: 5.5013x; 1.0053x over previous
import functools

import jax
import jax.numpy as jnp
import numpy as np
from jax import lax
from jax.experimental import pallas as pl
from jax.experimental.pallas import tpu as pltpu
from jax.experimental.pallas import tpu_sc as plsc

N = 10000
E = 160000
LIN = 256
OUT = 256
HEADS = 4
HO = HEADS * OUT
HID = 128
NPAD = 10240
NB = 512
GRID = NPAD // NB
NSUB = 32
DPS = NPAD // NSUB
RPW = DPS * (NSUB - 1) + 336
F32 = jnp.float32


def _leaky(x):
    return jnp.where(x > 0, x, 0.01 * x)


def _dot(a, b):
    return jnp.dot(a, b, preferred_element_type=F32)


def _enc_body(prop_r, cat_r, tw_r, de_r, wn, bn, wb, bb, wt, bt, wd, bd,
              w1, b1, x_r):
    un = _leaky(_dot(prop_r[...], wn[...]) + bn[...])
    ub = _leaky(_dot(cat_r[...], wb[...]) + bb[...])
    ut = _leaky(_dot(tw_r[...], wt[...]) + bt[...])
    ud = _leaky(_dot(de_r[...], wd[...]) + bd[...])
    u = jnp.concatenate([un, ub, ut, ud], axis=1)
    x_r[...] = _leaky(_dot(u, w1[...]) + b1[...])


def _row_spec(c):
    return pl.BlockSpec((NB, c), lambda i: (i, 0))


def _full_spec(shape):
    return pl.BlockSpec(shape, lambda i: tuple(0 for _ in shape))


def _encode(prop, cat, tw, de, p):
    wn, bn = p["in_num"]["W"], p["in_num"]["b"].reshape(1, -1)
    wb, bb = p["in_bool"]["W"], p["in_bool"]["b"].reshape(1, -1)
    wt, bt = p["in_tweet"]["W"], p["in_tweet"]["b"].reshape(1, -1)
    wd, bd = p["in_des"]["W"], p["in_des"]["b"].reshape(1, -1)
    w1, b1 = p["linear1"]["W"], p["linear1"]["b"].reshape(1, -1)
    ins = [prop, cat, tw, de, wn, bn, wb, bb, wt, bt, wd, bd, w1, b1]
    in_specs = [_row_spec(5), _row_spec(3), _row_spec(768), _row_spec(768)]
    in_specs += [_full_spec(a.shape) for a in ins[4:]]
    return pl.pallas_call(
        _enc_body, grid=(GRID,),
        in_specs=in_specs,
        out_specs=_row_spec(LIN),
        out_shape=jax.ShapeDtypeStruct((NPAD, LIN), F32),
    )(*ins)


def _qkvs_body(from_sems, x_or_sem0, *rest):
    if from_sems:
        sem0_r, sem1_r, coef_r = x_or_sem0, rest[0], rest[1]
        w = rest[2:18]
        outs = rest[18:]
        x2_r = outs[0]
        outs = outs[1:]
        c0 = coef_r[0:1, 0:1]
        c1 = coef_r[0:1, 1:2]
        xv = _leaky(sem0_r[...] * c0 + sem1_r[...] * c1)
        x2_r[...] = xv
    else:
        w = rest[:16]
        outs = rest[16:]
        xv = x_or_sem0[...]
    for t in range(2):
        wq, bq, wk, bk, wv, bv, ws, bs = w[8 * t:8 * t + 8]
        q_r, k_r, v_r, s_r = outs[4 * t:4 * t + 4]
        q_r[...] = _dot(xv, wq[...]) + bq[...]
        k_r[...] = _dot(xv, wk[...]) + bk[...]
        v_r[...] = _dot(xv, wv[...]) + bv[...]
        s_r[...] = _dot(xv, ws[...]) + bs[...]


def _qkvs_weights(rp):
    w = []
    for t in range(2):
        tp = rp["trans"][t]
        for name in ("q", "k", "v", "skip"):
            w.append(tp[name]["W"])
            w.append(tp[name]["b"].reshape(1, -1))
    return w


def _qkvs(x, rgt_params):
    w = _qkvs_weights(rgt_params)
    ins = [x] + w
    in_specs = [_row_spec(LIN)] + [_full_spec(a.shape) for a in w]
    out_shapes, out_specs = [], []
    for t in range(2):
        for c in (HO, HO, HO, OUT):
            out_shapes.append(jax.ShapeDtypeStruct((NPAD, c), F32))
            out_specs.append(_row_spec(c))
    return pl.pallas_call(
        functools.partial(_qkvs_body, False), grid=(GRID,),
        in_specs=in_specs, out_specs=out_specs, out_shape=out_shapes,
    )(*ins)


def _qkvs_from_sems(sem0, sem1, coef, rgt_params):
    w = _qkvs_weights(rgt_params)
    ins = [sem0, sem1, coef] + w
    in_specs = [_row_spec(LIN), _row_spec(LIN), _full_spec((8, 128))]
    in_specs += [_full_spec(a.shape) for a in w]
    out_shapes = [jax.ShapeDtypeStruct((NPAD, LIN), F32)]
    out_specs = [_row_spec(LIN)]
    for t in range(2):
        for c in (HO, HO, HO, OUT):
            out_shapes.append(jax.ShapeDtypeStruct((NPAD, c), F32))
            out_specs.append(_row_spec(c))
    return pl.pallas_call(
        functools.partial(_qkvs_body, True), grid=(GRID,),
        in_specs=in_specs, out_specs=out_specs, out_shape=out_shapes,
    )(*ins)


def _gate_body(x_r, c0_r, c1_r, s0_r, s1_r, wgu, wgx, bg,
               l1w0, l1b0, l2r0, l1w1, l1b1, l2r1,
               sem0_r, sem1_r, coef_r, wacc):
    i = pl.program_id(0)
    xv = x_r[...]
    sems = []
    for t in range(2):
        u = (c0_r, c1_r)[t][...] + (s0_r, s1_r)[t][...]
        a = jax.nn.sigmoid(_dot(u, wgu[...]) + _dot(xv, wgx[...]) + bg[...])
        sems.append(jnp.tanh(u) * a + xv * (1.0 - a))
    sem0_r[...] = sems[0]
    sem1_r[...] = sems[1]
    rows = i * NB + lax.broadcasted_iota(jnp.int32, (NB, 1), 0)
    rmask = rows < N
    rio = lax.broadcasted_iota(jnp.int32, (8, 128), 0)
    cio = lax.broadcasted_iota(jnp.int32, (8, 128), 1)
    wblk = jnp.zeros((8, 128), F32)
    for p_i, (l1w, l1b, l2r) in enumerate(((l1w0, l1b0, l2r0),
                                           (l1w1, l1b1, l2r1))):
        for t in range(2):
            t1 = jnp.tanh(_dot(sems[t], l1w[...]) + l1b[...])
            hn = jnp.sum(t1 * l2r[...], axis=1, keepdims=True)
            tot = jnp.sum(jnp.where(rmask, hn, 0.0))
            wblk = wblk + jnp.where((rio == p_i) & (cio == t), tot, 0.0)

    @pl.when(i == 0)
    def _():
        wacc[...] = wblk

    @pl.when(i > 0)
    def _():
        wacc[...] = wacc[...] + wblk

    @pl.when(i == GRID - 1)
    def _():
        w = wacc[...] / float(N)
        e = jnp.exp(w)
        den = jnp.sum(jnp.where(cio < 2, e, 0.0), axis=1, keepdims=True)
        r = jnp.where(cio < 2, e / den, 0.0)
        csum = jnp.sum(jnp.where(rio < 2, r, 0.0), axis=0, keepdims=True)
        coef_r[...] = jnp.broadcast_to(csum * 0.5, (8, 128))


def _gate(x, conv0, conv1, skip0, skip1, rgt_params):
    gw = rgt_params["gate"]["W"]
    wgu, wgx = gw[:LIN], gw[LIN:]
    bg = rgt_params["gate"]["b"].reshape(1, -1)
    sem = rgt_params["sem"]
    l1w0, l1b0 = sem[0]["l1"]["W"], sem[0]["l1"]["b"].reshape(1, -1)
    l2r0 = sem[0]["l2"]["W"].T
    l1w1, l1b1 = sem[1]["l1"]["W"], sem[1]["l1"]["b"].reshape(1, -1)
    l2r1 = sem[1]["l2"]["W"].T
    ins = [x, conv0, conv1, skip0, skip1, wgu, wgx, bg,
           l1w0, l1b0, l2r0, l1w1, l1b1, l2r1]
    in_specs = [_row_spec(LIN)] * 5 + [_full_spec(a.shape) for a in ins[5:]]
    out_shapes = [jax.ShapeDtypeStruct((NPAD, LIN), F32),
                  jax.ShapeDtypeStruct((NPAD, LIN), F32),
                  jax.ShapeDtypeStruct((8, 128), F32)]
    out_specs = [_row_spec(LIN), _row_spec(LIN), _full_spec((8, 128))]
    return pl.pallas_call(
        _gate_body, grid=(GRID,),
        in_specs=in_specs, out_specs=out_specs, out_shape=out_shapes,
        scratch_shapes=[pltpu.VMEM((8, 128), F32)],
    )(*ins)


def _final_body(sem0_r, sem1_r, coef_r, wo1, bo1, wo2, bo2, y_r):
    c0 = coef_r[0:1, 0:1]
    c1 = coef_r[0:1, 1:2]
    xv = _leaky(sem0_r[...] * c0 + sem1_r[...] * c1)
    h = _leaky(_dot(xv, wo1[...]) + bo1[...])
    y_r[...] = _dot(h, wo2[...]) + bo2[...]


def _final(sem0, sem1, coef, p):
    wo1, bo1 = p["out1"]["W"], p["out1"]["b"].reshape(1, -1)
    wo2 = jnp.pad(p["out2"]["W"], ((0, 0), (0, 126)))
    bo2 = jnp.pad(p["out2"]["b"].reshape(1, -1), ((0, 0), (0, 126)))
    ins = [sem0, sem1, coef, wo1, bo1, wo2, bo2]
    in_specs = [_row_spec(LIN), _row_spec(LIN), _full_spec((8, 128))]
    in_specs += [_full_spec(a.shape) for a in ins[3:]]
    return pl.pallas_call(
        _final_body, grid=(GRID,),
        in_specs=in_specs,
        out_specs=_row_spec(128),
        out_shape=jax.ShapeDtypeStruct((NPAD, 128), F32),
    )(*ins)


def _conv_body(rp_hbm, src_hbm, q0, k0, v0, q1, k1, v1, out_hbm,
               rp_v, win, idx, qbuf, kbuf, vbuf, acc, orow, sbuf, svec,
               semk, semv):
    wid = lax.axis_index("s") * 2 + lax.axis_index("c")
    d0 = wid * DPS
    ndst = jnp.minimum(DPS, N - d0)
    lanes = lax.iota(jnp.int32, 16)
    zero16 = jnp.zeros((16,), F32)
    minf = jnp.float32(-jnp.inf)
    for g in range(8):
        sbuf[pl.ds(16 * g, 16)] = zero16
    for t in range(2):
        pltpu.sync_copy(rp_hbm.at[pl.ds(t * RPW + d0, 336)],
                        rp_v.at[pl.ds(336 * t, 336)])

    for t, (qh, kh, vh) in enumerate(((q0, k0, v0), (q1, k1, v1))):

        def grp_body(jg, _, t=t, qh=qh, kh=kh, vh=vh):
            dg = d0 + 16 * jg
            pltpu.sync_copy(qh.at[pl.ds(dg, 16)], qbuf)
            nloc = jnp.minimum(16, ndst - 16 * jg)

            def node_body(jl, _2):
                j = 16 * jg + jl
                d = d0 + j
                rpvec = rp_v[pl.ds(336 * t + j, 16)]
                start = rpvec[0]
                end = rpvec[1]
                cnt = end - start
                for g in range(64):
                    acc[pl.ds(16 * g, 16)] = zero16
                nch = lax.div(cnt + 15, 16)

                def chunk_body(c, carry):
                    s0_ = start + 16 * c
                    valid = jnp.minimum(16, cnt - 16 * c)
                    a = lax.div(s0_, 8) * 8
                    off = s0_ - a
                    pltpu.sync_copy(src_hbm.at[pl.ds(a, 24)], win)
                    iv = win[pl.ds(off, 16)]
                    iv = jnp.where(lanes < valid, iv, 0)
                    idx[...] = iv
                    ck = pltpu.async_copy(kh.at[idx], kbuf, semk)
                    cv = pltpu.async_copy(vh.at[idx], vbuf, semv)
                    ck.wait()

                    def ebody(jj, ms):
                        dots = []
                        for h in range(4):
                            da = db = None
                            for g in range(8):
                                sl = pl.ds(256 * h + 16 * g, 16)
                                term = qbuf[jl, sl] * kbuf[jj, sl]
                                da = term if da is None else da + term
                            for g in range(8, 16):
                                sl = pl.ds(256 * h + 16 * g, 16)
                                term = qbuf[jl, sl] * kbuf[jj, sl]
                                db = term if db is None else db + term
                            dots.append(da + db)
                        for k in (8, 4, 2, 1):
                            for h in range(4):
                                sbuf[pl.ds(32 * h, 16)] = dots[h]
                            dots = [dots[h] + sbuf[pl.ds(32 * h + k, 16)]
                                    for h in range(4)]
                        s_ = [dots[h][0] * (1.0 / 16.0) for h in range(4)]
                        for h in range(4):
                            svec[jj, pl.ds(16 * h, 16)] = jnp.full(
                                (16,), s_[h], F32)
                        return tuple(jnp.maximum(ms[h], s_[h])
                                     for h in range(4))

                    newm = list(lax.fori_loop(0, valid, ebody, carry[:4]))
                    news = []
                    mfulls = []
                    for h in range(4):
                        scale = jnp.exp(jnp.full((16,), carry[h], F32)
                                        - jnp.full((16,), newm[h], F32))
                        news.append(carry[4 + h] * scale)
                        mfulls.append(jnp.full((16,), newm[h], F32))
                        for g in range(16):
                            sl = pl.ds(256 * h + 16 * g, 16)
                            acc[sl] = acc[sl] * scale
                    cv.wait()

                    def abody(jj, ps):
                        pvs = [jnp.exp(svec[jj, pl.ds(16 * h, 16)]
                                       - mfulls[h]) for h in range(4)]
                        for h in range(4):
                            for g in range(16):
                                sl = pl.ds(256 * h + 16 * g, 16)
                                plsc.addupdate(acc.at[sl],
                                               pvs[h] * vbuf[jj, sl])
                        return tuple(ps[h] + pvs[h] for h in range(4))

                    news = list(lax.fori_loop(0, valid, abody, tuple(news)))
                    return tuple(newm + news)

                fin = lax.fori_loop(0, nch, chunk_body,
                                    (minf,) * 4 + (zero16,) * 4)
                invs = [1.0 / (fin[4 + h] + 1e-16) for h in range(4)]
                for g in range(16):
                    o = zero16
                    for h in range(4):
                        o = o + acc[pl.ds(256 * h + 16 * g, 16)] * invs[h]
                    orow[pl.ds(16 * g, 16)] = o * 0.25
                pltpu.sync_copy(
                    orow, out_hbm.at[pl.ds((t * NPAD + d) * OUT, OUT)])
                return 0

            lax.fori_loop(0, nloc, node_body, 0)
            return 0

        lax.fori_loop(0, lax.div(ndst + 15, 16), grp_body, 0)


def _sc_conv(rp2, ssrcp, q0, k0, v0, q1, k1, v1):
    mesh = plsc.VectorSubcoreMesh(core_axis_name="c", subcore_axis_name="s")
    kfn = pl.kernel(
        _conv_body,
        out_type=jax.ShapeDtypeStruct((2 * NPAD * OUT,), F32),
        mesh=mesh,
        scratch_types=[
            pltpu.VMEM((672,), jnp.int32),
            pltpu.VMEM((24,), jnp.int32),
            pltpu.VMEM((16,), jnp.int32),
            pltpu.VMEM((16, HO), F32),
            pltpu.VMEM((16, HO), F32),
            pltpu.VMEM((16, HO), F32),
            pltpu.VMEM((HO,), F32),
            pltpu.VMEM((OUT,), F32),
            pltpu.VMEM((128,), F32),
            pltpu.VMEM((16, 64), F32),
            pltpu.SemaphoreType.DMA,
            pltpu.SemaphoreType.DMA,
        ],
    )
    out = kfn(rp2, ssrcp, q0, k0, v0, q1, k1, v1)
    return out.reshape(2, NPAD, OUT)


def kernel(cat_features, prop_features, tweet_features, des_features,
           edge_index, edge_type, params):
    src = edge_index[0].astype(jnp.int32)
    dst = edge_index[1].astype(jnp.int32)
    et = edge_type.astype(jnp.int32)
    key = et * N + dst
    order = jnp.argsort(key)
    skey = jnp.take(key, order)
    ssrc = jnp.take(src, order)
    rp = jnp.searchsorted(skey, jnp.arange(2 * N + 1, dtype=jnp.int32))
    rp = rp.astype(jnp.int32)
    rp2 = jnp.stack([rp[:N + 1], rp[N:2 * N + 1]])
    rp2 = jnp.pad(rp2, ((0, 0), (0, RPW - (N + 1))), mode="edge").reshape(-1)
    ssrcp = jnp.pad(ssrc, (0, 24))

    x = _encode(prop_features, cat_features, tweet_features, des_features,
                params)
    sem0 = sem1 = coef = None
    for li, rgt in enumerate((params["rgt1"], params["rgt2"])):
        if li == 0:
            q0, k0, v0, s0, q1, k1, v1, s1 = _qkvs(x, rgt)
        else:
            x, q0, k0, v0, s0, q1, k1, v1, s1 = _qkvs_from_sems(
                sem0, sem1, coef, rgt)
        conv = _sc_conv(rp2, ssrcp, q0, k0, v0, q1, k1, v1)
        sem0, sem1, coef = _gate(x, conv[0], conv[1], s0, s1, rgt)
    y = _final(sem0, sem1, coef, params)
    return y[:N, :2]


# speedup vs baseline: 5.6074x; 1.0193x over previous
import functools

import jax
import jax.numpy as jnp
import numpy as np
from jax import lax
from jax.experimental import pallas as pl
from jax.experimental.pallas import tpu as pltpu
from jax.experimental.pallas import tpu_sc as plsc

N = 10000
E = 160000
LIN = 256
OUT = 256
HEADS = 4
HO = HEADS * OUT
HID = 128
NPAD = 10240
NB = 512
GRID = NPAD // NB
NSUB = 32
DPS = NPAD // NSUB
RPW = DPS * (NSUB - 1) + 336
F32 = jnp.float32


def _leaky(x):
    return jnp.where(x > 0, x, 0.01 * x)


def _dot(a, b):
    return jnp.dot(a, b, preferred_element_type=F32)


def _enc_body(prop_r, cat_r, tw_r, de_r, wn, bn, wb, bb, wt, bt, wd, bd,
              w1, b1, x_r):
    un = _leaky(_dot(prop_r[...], wn[...]) + bn[...])
    ub = _leaky(_dot(cat_r[...], wb[...]) + bb[...])
    ut = _leaky(_dot(tw_r[...], wt[...]) + bt[...])
    ud = _leaky(_dot(de_r[...], wd[...]) + bd[...])
    u = jnp.concatenate([un, ub, ut, ud], axis=1)
    x_r[...] = _leaky(_dot(u, w1[...]) + b1[...])


def _row_spec(c):
    return pl.BlockSpec((NB, c), lambda i: (i, 0))


def _full_spec(shape):
    return pl.BlockSpec(shape, lambda i: tuple(0 for _ in shape))


def _encode(prop, cat, tw, de, p):
    wn, bn = p["in_num"]["W"], p["in_num"]["b"].reshape(1, -1)
    wb, bb = p["in_bool"]["W"], p["in_bool"]["b"].reshape(1, -1)
    wt, bt = p["in_tweet"]["W"], p["in_tweet"]["b"].reshape(1, -1)
    wd, bd = p["in_des"]["W"], p["in_des"]["b"].reshape(1, -1)
    w1, b1 = p["linear1"]["W"], p["linear1"]["b"].reshape(1, -1)
    ins = [prop, cat, tw, de, wn, bn, wb, bb, wt, bt, wd, bd, w1, b1]
    in_specs = [_row_spec(5), _row_spec(3), _row_spec(768), _row_spec(768)]
    in_specs += [_full_spec(a.shape) for a in ins[4:]]
    return pl.pallas_call(
        _enc_body, grid=(GRID,),
        in_specs=in_specs,
        out_specs=_row_spec(LIN),
        out_shape=jax.ShapeDtypeStruct((NPAD, LIN), F32),
    )(*ins)


def _qkvs_body(from_sems, x_or_sem0, *rest):
    if from_sems:
        sem0_r, sem1_r, coef_r = x_or_sem0, rest[0], rest[1]
        w = rest[2:18]
        outs = rest[18:]
        x2_r = outs[0]
        outs = outs[1:]
        c0 = coef_r[0:1, 0:1]
        c1 = coef_r[0:1, 1:2]
        xv = _leaky(sem0_r[...] * c0 + sem1_r[...] * c1)
        x2_r[...] = xv
    else:
        w = rest[:16]
        outs = rest[16:]
        xv = x_or_sem0[...]
    for t in range(2):
        wq, bq, wk, bk, wv, bv, ws, bs = w[8 * t:8 * t + 8]
        q_r, kv_r, s_r = outs[3 * t:3 * t + 3]
        q_r[...] = _dot(xv, wq[...]) + bq[...]
        kv_r[:, 0:HO] = _dot(xv, wk[...]) + bk[...]
        kv_r[:, HO:2 * HO] = _dot(xv, wv[...]) + bv[...]
        s_r[...] = _dot(xv, ws[...]) + bs[...]


def _qkvs_weights(rp):
    w = []
    for t in range(2):
        tp = rp["trans"][t]
        for name in ("q", "k", "v", "skip"):
            w.append(tp[name]["W"])
            w.append(tp[name]["b"].reshape(1, -1))
    return w


def _qkvs(x, rgt_params):
    w = _qkvs_weights(rgt_params)
    ins = [x] + w
    in_specs = [_row_spec(LIN)] + [_full_spec(a.shape) for a in w]
    out_shapes, out_specs = [], []
    for t in range(2):
        for c in (HO, 2 * HO, OUT):
            out_shapes.append(jax.ShapeDtypeStruct((NPAD, c), F32))
            out_specs.append(_row_spec(c))
    return pl.pallas_call(
        functools.partial(_qkvs_body, False), grid=(GRID,),
        in_specs=in_specs, out_specs=out_specs, out_shape=out_shapes,
    )(*ins)


def _qkvs_from_sems(sem0, sem1, coef, rgt_params):
    w = _qkvs_weights(rgt_params)
    ins = [sem0, sem1, coef] + w
    in_specs = [_row_spec(LIN), _row_spec(LIN), _full_spec((8, 128))]
    in_specs += [_full_spec(a.shape) for a in w]
    out_shapes = [jax.ShapeDtypeStruct((NPAD, LIN), F32)]
    out_specs = [_row_spec(LIN)]
    for t in range(2):
        for c in (HO, 2 * HO, OUT):
            out_shapes.append(jax.ShapeDtypeStruct((NPAD, c), F32))
            out_specs.append(_row_spec(c))
    return pl.pallas_call(
        functools.partial(_qkvs_body, True), grid=(GRID,),
        in_specs=in_specs, out_specs=out_specs, out_shape=out_shapes,
    )(*ins)


def _gate_body(x_r, c0_r, c1_r, s0_r, s1_r, wgu, wgx, bg,
               l1w0, l1b0, l2r0, l1w1, l1b1, l2r1,
               sem0_r, sem1_r, coef_r, wacc):
    i = pl.program_id(0)
    xv = x_r[...]
    sems = []
    for t in range(2):
        u = (c0_r, c1_r)[t][...] + (s0_r, s1_r)[t][...]
        a = jax.nn.sigmoid(_dot(u, wgu[...]) + _dot(xv, wgx[...]) + bg[...])
        sems.append(jnp.tanh(u) * a + xv * (1.0 - a))
    sem0_r[...] = sems[0]
    sem1_r[...] = sems[1]
    rows = i * NB + lax.broadcasted_iota(jnp.int32, (NB, 1), 0)
    rmask = rows < N
    rio = lax.broadcasted_iota(jnp.int32, (8, 128), 0)
    cio = lax.broadcasted_iota(jnp.int32, (8, 128), 1)
    wblk = jnp.zeros((8, 128), F32)
    for p_i, (l1w, l1b, l2r) in enumerate(((l1w0, l1b0, l2r0),
                                           (l1w1, l1b1, l2r1))):
        for t in range(2):
            t1 = jnp.tanh(_dot(sems[t], l1w[...]) + l1b[...])
            hn = jnp.sum(t1 * l2r[...], axis=1, keepdims=True)
            tot = jnp.sum(jnp.where(rmask, hn, 0.0))
            wblk = wblk + jnp.where((rio == p_i) & (cio == t), tot, 0.0)

    @pl.when(i == 0)
    def _():
        wacc[...] = wblk

    @pl.when(i > 0)
    def _():
        wacc[...] = wacc[...] + wblk

    @pl.when(i == GRID - 1)
    def _():
        w = wacc[...] / float(N)
        e = jnp.exp(w)
        den = jnp.sum(jnp.where(cio < 2, e, 0.0), axis=1, keepdims=True)
        r = jnp.where(cio < 2, e / den, 0.0)
        csum = jnp.sum(jnp.where(rio < 2, r, 0.0), axis=0, keepdims=True)
        coef_r[...] = jnp.broadcast_to(csum * 0.5, (8, 128))


def _gate(x, conv0, conv1, skip0, skip1, rgt_params):
    gw = rgt_params["gate"]["W"]
    wgu, wgx = gw[:LIN], gw[LIN:]
    bg = rgt_params["gate"]["b"].reshape(1, -1)
    sem = rgt_params["sem"]
    l1w0, l1b0 = sem[0]["l1"]["W"], sem[0]["l1"]["b"].reshape(1, -1)
    l2r0 = sem[0]["l2"]["W"].T
    l1w1, l1b1 = sem[1]["l1"]["W"], sem[1]["l1"]["b"].reshape(1, -1)
    l2r1 = sem[1]["l2"]["W"].T
    ins = [x, conv0, conv1, skip0, skip1, wgu, wgx, bg,
           l1w0, l1b0, l2r0, l1w1, l1b1, l2r1]
    in_specs = [_row_spec(LIN)] * 5 + [_full_spec(a.shape) for a in ins[5:]]
    out_shapes = [jax.ShapeDtypeStruct((NPAD, LIN), F32),
                  jax.ShapeDtypeStruct((NPAD, LIN), F32),
                  jax.ShapeDtypeStruct((8, 128), F32)]
    out_specs = [_row_spec(LIN), _row_spec(LIN), _full_spec((8, 128))]
    return pl.pallas_call(
        _gate_body, grid=(GRID,),
        in_specs=in_specs, out_specs=out_specs, out_shape=out_shapes,
        scratch_shapes=[pltpu.VMEM((8, 128), F32)],
    )(*ins)


def _final_body(sem0_r, sem1_r, coef_r, wo1, bo1, wo2, bo2, y_r):
    c0 = coef_r[0:1, 0:1]
    c1 = coef_r[0:1, 1:2]
    xv = _leaky(sem0_r[...] * c0 + sem1_r[...] * c1)
    h = _leaky(_dot(xv, wo1[...]) + bo1[...])
    y_r[...] = _dot(h, wo2[...]) + bo2[...]


def _final(sem0, sem1, coef, p):
    wo1, bo1 = p["out1"]["W"], p["out1"]["b"].reshape(1, -1)
    wo2 = jnp.pad(p["out2"]["W"], ((0, 0), (0, 126)))
    bo2 = jnp.pad(p["out2"]["b"].reshape(1, -1), ((0, 0), (0, 126)))
    ins = [sem0, sem1, coef, wo1, bo1, wo2, bo2]
    in_specs = [_row_spec(LIN), _row_spec(LIN), _full_spec((8, 128))]
    in_specs += [_full_spec(a.shape) for a in ins[3:]]
    return pl.pallas_call(
        _final_body, grid=(GRID,),
        in_specs=in_specs,
        out_specs=_row_spec(128),
        out_shape=jax.ShapeDtypeStruct((NPAD, 128), F32),
    )(*ins)


def _conv_body(rp_hbm, src_hbm, q0, kv0, q1, kv1, out_hbm,
               rp_v, win, idx, qbuf, kvbuf, acc, orow, sbuf, svec,
               semk, semo):
    wid = lax.axis_index("s") * 2 + lax.axis_index("c")
    d0 = wid * DPS
    ndst = jnp.minimum(DPS, N - d0)
    lanes = lax.iota(jnp.int32, 16)
    zero16 = jnp.zeros((16,), F32)
    minf = jnp.float32(-jnp.inf)
    for g in range(8):
        sbuf[pl.ds(16 * g, 16)] = zero16
    for t in range(2):
        pltpu.sync_copy(rp_hbm.at[pl.ds(t * RPW + d0, 336)],
                        rp_v.at[pl.ds(336 * t, 336)])

    for t, (qh, kvh) in enumerate(((q0, kv0), (q1, kv1))):

        def grp_body(jg, gcarry, t=t, qh=qh, kvh=kvh):
            dg = d0 + 16 * jg
            pltpu.sync_copy(qh.at[pl.ds(dg, 16)], qbuf)
            nloc = jnp.minimum(16, ndst - 16 * jg)

            def node_body(jl, ncarry):
                win_base0, issued0 = ncarry
                j = 16 * jg + jl
                d = d0 + j
                rpvec = rp_v[pl.ds(336 * t + j, 16)]
                start = rpvec[0]
                end = rpvec[1]
                cnt = end - start
                for g in range(64):
                    acc[pl.ds(16 * g, 16)] = zero16
                nch = lax.div(cnt + 15, 16)

                def chunk_body(c, carry):
                    s0_ = start + 16 * c
                    valid = jnp.minimum(16, cnt - 16 * c)
                    wb = carry[8]

                    def _refill(_):
                        nb_ = lax.div(s0_, 8) * 8
                        pltpu.sync_copy(src_hbm.at[pl.ds(nb_, 264)], win)
                        return nb_

                    wb = lax.cond(s0_ - wb > 248, _refill,
                                  lambda _: wb, None)
                    off = s0_ - wb
                    iv = win[pl.ds(off, 16)]
                    iv = jnp.where(lanes < valid, iv, 0)
                    idx[...] = iv
                    ck = pltpu.async_copy(kvh.at[idx], kvbuf, semk)
                    ck.wait()

                    def ebody(jj, ms):
                        dots = []
                        for h in range(4):
                            da = db = None
                            for g in range(8):
                                sl = pl.ds(256 * h + 16 * g, 16)
                                term = qbuf[jl, sl] * kvbuf[jj, sl]
                                da = term if da is None else da + term
                            for g in range(8, 16):
                                sl = pl.ds(256 * h + 16 * g, 16)
                                term = qbuf[jl, sl] * kvbuf[jj, sl]
                                db = term if db is None else db + term
                            dots.append(da + db)
                        for k in (8, 4, 2, 1):
                            for h in range(4):
                                sbuf[pl.ds(32 * h, 16)] = dots[h]
                            dots = [dots[h] + sbuf[pl.ds(32 * h + k, 16)]
                                    for h in range(4)]
                        s_ = [dots[h][0] * (1.0 / 16.0) for h in range(4)]
                        for h in range(4):
                            svec[jj, pl.ds(16 * h, 16)] = jnp.full(
                                (16,), s_[h], F32)
                        return tuple(jnp.maximum(ms[h], s_[h])
                                     for h in range(4))

                    newm = list(lax.fori_loop(0, valid, ebody, carry[:4]))
                    news = []
                    mfulls = []
                    for h in range(4):
                        scale = jnp.exp(jnp.full((16,), carry[h], F32)
                                        - jnp.full((16,), newm[h], F32))
                        news.append(carry[4 + h] * scale)
                        mfulls.append(jnp.full((16,), newm[h], F32))
                        for g in range(16):
                            sl = pl.ds(256 * h + 16 * g, 16)
                            acc[sl] = acc[sl] * scale

                    def abody(jj, ps):
                        pvs = [jnp.exp(svec[jj, pl.ds(16 * h, 16)]
                                       - mfulls[h]) for h in range(4)]
                        for h in range(4):
                            for g in range(16):
                                off_ = 256 * h + 16 * g
                                plsc.addupdate(
                                    acc.at[pl.ds(off_, 16)],
                                    pvs[h] * kvbuf[jj, pl.ds(HO + off_, 16)])
                        return tuple(ps[h] + pvs[h] for h in range(4))

                    news = list(lax.fori_loop(0, valid, abody, tuple(news)))
                    return tuple(newm + news) + (wb,)

                fin = lax.fori_loop(0, nch, chunk_body,
                                    (minf,) * 4 + (zero16,) * 4 + (win_base0,))
                invs = [1.0 / (fin[4 + h] + 1e-16) for h in range(4)]

                @pl.when(issued0 == 1)
                def _():
                    pltpu.make_async_copy(
                        orow, out_hbm.at[pl.ds(0, OUT)], semo).wait()

                for g in range(16):
                    o = zero16
                    for h in range(4):
                        o = o + acc[pl.ds(256 * h + 16 * g, 16)] * invs[h]
                    orow[pl.ds(16 * g, 16)] = o * 0.25
                pltpu.async_copy(
                    orow, out_hbm.at[pl.ds((t * NPAD + d) * OUT, OUT)], semo)
                return (fin[8], jnp.int32(1))

            return lax.fori_loop(0, nloc, node_body, gcarry)

        gfin = lax.fori_loop(0, lax.div(ndst + 15, 16), grp_body,
                             (jnp.int32(-(1 << 30)), jnp.int32(0)))

        @pl.when(gfin[1] == 1)
        def _():
            pltpu.make_async_copy(
                orow, out_hbm.at[pl.ds(0, OUT)], semo).wait()


def _sc_conv(rp2, ssrcp, q0, kv0, q1, kv1):
    mesh = plsc.VectorSubcoreMesh(core_axis_name="c", subcore_axis_name="s")
    kfn = pl.kernel(
        _conv_body,
        out_type=jax.ShapeDtypeStruct((2 * NPAD * OUT,), F32),
        mesh=mesh,
        scratch_types=[
            pltpu.VMEM((672,), jnp.int32),
            pltpu.VMEM((264,), jnp.int32),
            pltpu.VMEM((16,), jnp.int32),
            pltpu.VMEM((16, HO), F32),
            pltpu.VMEM((16, 2 * HO), F32),
            pltpu.VMEM((HO,), F32),
            pltpu.VMEM((OUT,), F32),
            pltpu.VMEM((128,), F32),
            pltpu.VMEM((16, 64), F32),
            pltpu.SemaphoreType.DMA,
            pltpu.SemaphoreType.DMA,
        ],
    )
    out = kfn(rp2, ssrcp, q0, kv0, q1, kv1)
    return out.reshape(2, NPAD, OUT)


def kernel(cat_features, prop_features, tweet_features, des_features,
           edge_index, edge_type, params):
    src = edge_index[0].astype(jnp.int32)
    dst = edge_index[1].astype(jnp.int32)
    et = edge_type.astype(jnp.int32)
    key = et * N + dst
    order = jnp.argsort(key)
    skey = jnp.take(key, order)
    ssrc = jnp.take(src, order)
    rp = jnp.searchsorted(skey, jnp.arange(2 * N + 1, dtype=jnp.int32))
    rp = rp.astype(jnp.int32)
    rp2 = jnp.stack([rp[:N + 1], rp[N:2 * N + 1]])
    rp2 = jnp.pad(rp2, ((0, 0), (0, RPW - (N + 1))), mode="edge").reshape(-1)
    ssrcp = jnp.pad(ssrc, (0, 272))

    x = _encode(prop_features, cat_features, tweet_features, des_features,
                params)
    sem0 = sem1 = coef = None
    for li, rgt in enumerate((params["rgt1"], params["rgt2"])):
        if li == 0:
            q0, kv0, s0, q1, kv1, s1 = _qkvs(x, rgt)
        else:
            x, q0, kv0, s0, q1, kv1, s1 = _qkvs_from_sems(
                sem0, sem1, coef, rgt)
        conv = _sc_conv(rp2, ssrcp, q0, kv0, q1, kv1)
        sem0, sem1, coef = _gate(x, conv[0], conv[1], s0, s1, rgt)
    y = _final(sem0, sem1, coef, params)
    return y[:N, :2]


# speedup vs baseline: 5.6078x; 1.0001x over previous
import functools

import jax
import jax.numpy as jnp
import numpy as np
from jax import lax
from jax.experimental import pallas as pl
from jax.experimental.pallas import tpu as pltpu
from jax.experimental.pallas import tpu_sc as plsc

N = 10000
E = 160000
LIN = 256
OUT = 256
HEADS = 4
HO = HEADS * OUT
HID = 128
NPAD = 10240
NB = 512
GRID = NPAD // NB
NSUB = 32
DPS = NPAD // NSUB
RPW = DPS * (NSUB - 1) + 336
F32 = jnp.float32


def _leaky(x):
    return jnp.where(x > 0, x, 0.01 * x)


def _dot(a, b):
    return jnp.dot(a, b, preferred_element_type=F32)


def _enc_body(prop_r, cat_r, tw_r, de_r, wn, bn, wb, bb, wt, bt, wd, bd,
              w1, b1, x_r):
    un = _leaky(_dot(prop_r[...], wn[...]) + bn[...])
    ub = _leaky(_dot(cat_r[...], wb[...]) + bb[...])
    ut = _leaky(_dot(tw_r[...], wt[...]) + bt[...])
    ud = _leaky(_dot(de_r[...], wd[...]) + bd[...])
    u = jnp.concatenate([un, ub, ut, ud], axis=1)
    x_r[...] = _leaky(_dot(u, w1[...]) + b1[...])


def _row_spec(c):
    return pl.BlockSpec((NB, c), lambda i: (i, 0))


def _full_spec(shape):
    return pl.BlockSpec(shape, lambda i: tuple(0 for _ in shape))


def _encode(prop, cat, tw, de, p):
    wn, bn = p["in_num"]["W"], p["in_num"]["b"].reshape(1, -1)
    wb, bb = p["in_bool"]["W"], p["in_bool"]["b"].reshape(1, -1)
    wt, bt = p["in_tweet"]["W"], p["in_tweet"]["b"].reshape(1, -1)
    wd, bd = p["in_des"]["W"], p["in_des"]["b"].reshape(1, -1)
    w1, b1 = p["linear1"]["W"], p["linear1"]["b"].reshape(1, -1)
    ins = [prop, cat, tw, de, wn, bn, wb, bb, wt, bt, wd, bd, w1, b1]
    in_specs = [_row_spec(5), _row_spec(3), _row_spec(768), _row_spec(768)]
    in_specs += [_full_spec(a.shape) for a in ins[4:]]
    return pl.pallas_call(
        _enc_body, grid=(GRID,),
        in_specs=in_specs,
        out_specs=_row_spec(LIN),
        out_shape=jax.ShapeDtypeStruct((NPAD, LIN), F32),
    )(*ins)


def _qkvs_body(from_sems, x_or_sem0, *rest):
    if from_sems:
        sem0_r, sem1_r, coef_r = x_or_sem0, rest[0], rest[1]
        w = rest[2:18]
        outs = rest[18:]
        x2_r = outs[0]
        outs = outs[1:]
        c0 = coef_r[0:1, 0:1]
        c1 = coef_r[0:1, 1:2]
        xv = _leaky(sem0_r[...] * c0 + sem1_r[...] * c1)
        x2_r[...] = xv
    else:
        w = rest[:16]
        outs = rest[16:]
        xv = x_or_sem0[...]
    for t in range(2):
        wq, bq, wk, bk, wv, bv, ws, bs = w[8 * t:8 * t + 8]
        q_r, kv_r, s_r = outs[3 * t:3 * t + 3]
        q_r[...] = _dot(xv, wq[...]) + bq[...]
        kv_r[:, 0:HO] = _dot(xv, wk[...]) + bk[...]
        kv_r[:, HO:2 * HO] = _dot(xv, wv[...]) + bv[...]
        s_r[...] = _dot(xv, ws[...]) + bs[...]


def _qkvs_weights(rp):
    w = []
    for t in range(2):
        tp = rp["trans"][t]
        for name in ("q", "k", "v", "skip"):
            w.append(tp[name]["W"])
            w.append(tp[name]["b"].reshape(1, -1))
    return w


def _qkvs(x, rgt_params):
    w = _qkvs_weights(rgt_params)
    ins = [x] + w
    in_specs = [_row_spec(LIN)] + [_full_spec(a.shape) for a in w]
    out_shapes, out_specs = [], []
    for t in range(2):
        for c in (HO, 2 * HO, OUT):
            out_shapes.append(jax.ShapeDtypeStruct((NPAD, c), F32))
            out_specs.append(_row_spec(c))
    return pl.pallas_call(
        functools.partial(_qkvs_body, False), grid=(GRID,),
        in_specs=in_specs, out_specs=out_specs, out_shape=out_shapes,
    )(*ins)


def _qkvs_from_sems(sem0, sem1, coef, rgt_params):
    w = _qkvs_weights(rgt_params)
    ins = [sem0, sem1, coef] + w
    in_specs = [_row_spec(LIN), _row_spec(LIN), _full_spec((8, 128))]
    in_specs += [_full_spec(a.shape) for a in w]
    out_shapes = [jax.ShapeDtypeStruct((NPAD, LIN), F32)]
    out_specs = [_row_spec(LIN)]
    for t in range(2):
        for c in (HO, 2 * HO, OUT):
            out_shapes.append(jax.ShapeDtypeStruct((NPAD, c), F32))
            out_specs.append(_row_spec(c))
    return pl.pallas_call(
        functools.partial(_qkvs_body, True), grid=(GRID,),
        in_specs=in_specs, out_specs=out_specs, out_shape=out_shapes,
    )(*ins)


def _gate_body(x_r, c0_r, c1_r, s0_r, s1_r, wgu, wgx, bg,
               l1w0, l1b0, l2r0, l1w1, l1b1, l2r1,
               sem0_r, sem1_r, coef_r, wacc):
    i = pl.program_id(0)
    xv = x_r[...]
    sems = []
    for t in range(2):
        u = (c0_r, c1_r)[t][...] + (s0_r, s1_r)[t][...]
        a = jax.nn.sigmoid(_dot(u, wgu[...]) + _dot(xv, wgx[...]) + bg[...])
        sems.append(jnp.tanh(u) * a + xv * (1.0 - a))
    sem0_r[...] = sems[0]
    sem1_r[...] = sems[1]
    rows = i * NB + lax.broadcasted_iota(jnp.int32, (NB, 1), 0)
    rmask = rows < N
    rio = lax.broadcasted_iota(jnp.int32, (8, 128), 0)
    cio = lax.broadcasted_iota(jnp.int32, (8, 128), 1)
    wblk = jnp.zeros((8, 128), F32)
    for p_i, (l1w, l1b, l2r) in enumerate(((l1w0, l1b0, l2r0),
                                           (l1w1, l1b1, l2r1))):
        for t in range(2):
            t1 = jnp.tanh(_dot(sems[t], l1w[...]) + l1b[...])
            hn = jnp.sum(t1 * l2r[...], axis=1, keepdims=True)
            tot = jnp.sum(jnp.where(rmask, hn, 0.0))
            wblk = wblk + jnp.where((rio == p_i) & (cio == t), tot, 0.0)

    @pl.when(i == 0)
    def _():
        wacc[...] = wblk

    @pl.when(i > 0)
    def _():
        wacc[...] = wacc[...] + wblk

    @pl.when(i == GRID - 1)
    def _():
        w = wacc[...] / float(N)
        e = jnp.exp(w)
        den = jnp.sum(jnp.where(cio < 2, e, 0.0), axis=1, keepdims=True)
        r = jnp.where(cio < 2, e / den, 0.0)
        csum = jnp.sum(jnp.where(rio < 2, r, 0.0), axis=0, keepdims=True)
        coef_r[...] = jnp.broadcast_to(csum * 0.5, (8, 128))


def _gate(x, conv0, conv1, skip0, skip1, rgt_params):
    gw = rgt_params["gate"]["W"]
    wgu, wgx = gw[:LIN], gw[LIN:]
    bg = rgt_params["gate"]["b"].reshape(1, -1)
    sem = rgt_params["sem"]
    l1w0, l1b0 = sem[0]["l1"]["W"], sem[0]["l1"]["b"].reshape(1, -1)
    l2r0 = sem[0]["l2"]["W"].T
    l1w1, l1b1 = sem[1]["l1"]["W"], sem[1]["l1"]["b"].reshape(1, -1)
    l2r1 = sem[1]["l2"]["W"].T
    ins = [x, conv0, conv1, skip0, skip1, wgu, wgx, bg,
           l1w0, l1b0, l2r0, l1w1, l1b1, l2r1]
    in_specs = [_row_spec(LIN)] * 5 + [_full_spec(a.shape) for a in ins[5:]]
    out_shapes = [jax.ShapeDtypeStruct((NPAD, LIN), F32),
                  jax.ShapeDtypeStruct((NPAD, LIN), F32),
                  jax.ShapeDtypeStruct((8, 128), F32)]
    out_specs = [_row_spec(LIN), _row_spec(LIN), _full_spec((8, 128))]
    return pl.pallas_call(
        _gate_body, grid=(GRID,),
        in_specs=in_specs, out_specs=out_specs, out_shape=out_shapes,
        scratch_shapes=[pltpu.VMEM((8, 128), F32)],
    )(*ins)


def _final_body(sem0_r, sem1_r, coef_r, wo1, bo1, wo2, bo2, y_r):
    c0 = coef_r[0:1, 0:1]
    c1 = coef_r[0:1, 1:2]
    xv = _leaky(sem0_r[...] * c0 + sem1_r[...] * c1)
    h = _leaky(_dot(xv, wo1[...]) + bo1[...])
    y_r[...] = _dot(h, wo2[...]) + bo2[...]


def _final(sem0, sem1, coef, p):
    wo1, bo1 = p["out1"]["W"], p["out1"]["b"].reshape(1, -1)
    wo2 = jnp.pad(p["out2"]["W"], ((0, 0), (0, 126)))
    bo2 = jnp.pad(p["out2"]["b"].reshape(1, -1), ((0, 0), (0, 126)))
    ins = [sem0, sem1, coef, wo1, bo1, wo2, bo2]
    in_specs = [_row_spec(LIN), _row_spec(LIN), _full_spec((8, 128))]
    in_specs += [_full_spec(a.shape) for a in ins[3:]]
    return pl.pallas_call(
        _final_body, grid=(GRID,),
        in_specs=in_specs,
        out_specs=_row_spec(128),
        out_shape=jax.ShapeDtypeStruct((NPAD, 128), F32),
    )(*ins)


def _conv_body(rp_hbm, src_hbm, q0, kv0, q1, kv1, out_hbm,
               rp_v, win, idx, qbuf, kvbuf, acc, orow, sbuf, svec,
               semk, semo):
    wid = lax.axis_index("s") * 2 + lax.axis_index("c")
    d0 = wid * DPS
    ndst = jnp.minimum(DPS, N - d0)
    lanes = lax.iota(jnp.int32, 16)
    zero16 = jnp.zeros((16,), F32)
    minf = jnp.float32(-jnp.inf)
    for g in range(16):
        sbuf[pl.ds(16 * g, 16)] = zero16
    for t in range(2):
        pltpu.sync_copy(rp_hbm.at[pl.ds(t * RPW + d0, 336)],
                        rp_v.at[pl.ds(336 * t, 336)])

    for t, (qh, kvh) in enumerate(((q0, kv0), (q1, kv1))):

        def grp_body(jg, gcarry, t=t, qh=qh, kvh=kvh):
            dg = d0 + 16 * jg
            pltpu.sync_copy(qh.at[pl.ds(dg, 16)], qbuf)
            nloc = jnp.minimum(16, ndst - 16 * jg)

            def node_body(jl, ncarry):
                win_base0, issued0 = ncarry
                j = 16 * jg + jl
                d = d0 + j
                rpvec = rp_v[pl.ds(336 * t + j, 16)]
                start = rpvec[0]
                end = rpvec[1]
                cnt = end - start
                for g in range(64):
                    acc[pl.ds(16 * g, 16)] = zero16
                nch = lax.div(cnt + 15, 16)

                def chunk_body(c, carry):
                    s0_ = start + 16 * c
                    valid = jnp.minimum(16, cnt - 16 * c)
                    wb = carry[8]

                    def _refill(_):
                        nb_ = lax.div(s0_, 8) * 8
                        pltpu.sync_copy(src_hbm.at[pl.ds(nb_, 264)], win)
                        return nb_

                    wb = lax.cond(s0_ - wb > 248, _refill,
                                  lambda _: wb, None)
                    off = s0_ - wb
                    iv = win[pl.ds(off, 16)]
                    iv = jnp.where(lanes < valid, iv, 0)
                    idx[...] = iv
                    ck = pltpu.async_copy(kvh.at[idx], kvbuf, semk)
                    ck.wait()

                    @plsc.parallel_loop(0, valid, unroll=2,
                                        carry=tuple(carry[:4]))
                    def newm(jj, ms):
                        rb = 128 * (jj & 1)
                        dots = []
                        for h in range(4):
                            da = db = None
                            for g in range(8):
                                sl = pl.ds(256 * h + 16 * g, 16)
                                term = qbuf[jl, sl] * kvbuf[jj, sl]
                                da = term if da is None else da + term
                            for g in range(8, 16):
                                sl = pl.ds(256 * h + 16 * g, 16)
                                term = qbuf[jl, sl] * kvbuf[jj, sl]
                                db = term if db is None else db + term
                            dots.append(da + db)
                        for k in (8, 4, 2, 1):
                            for h in range(4):
                                sbuf[pl.ds(rb + 32 * h, 16)] = dots[h]
                            dots = [dots[h] + sbuf[pl.ds(rb + 32 * h + k, 16)]
                                    for h in range(4)]
                        s_ = [dots[h][0] * (1.0 / 16.0) for h in range(4)]
                        for h in range(4):
                            svec[jj, pl.ds(16 * h, 16)] = jnp.full(
                                (16,), s_[h], F32)
                        return tuple(jnp.maximum(ms[h], s_[h])
                                     for h in range(4))

                    newm = list(newm)
                    news = []
                    mfulls = []
                    for h in range(4):
                        scale = jnp.exp(jnp.full((16,), carry[h], F32)
                                        - jnp.full((16,), newm[h], F32))
                        news.append(carry[4 + h] * scale)
                        mfulls.append(jnp.full((16,), newm[h], F32))
                        for g in range(16):
                            sl = pl.ds(256 * h + 16 * g, 16)
                            acc[sl] = acc[sl] * scale

                    @plsc.parallel_loop(0, valid, unroll=2,
                                        carry=tuple(news))
                    def news(jj, ps):
                        pvs = [jnp.exp(svec[jj, pl.ds(16 * h, 16)]
                                       - mfulls[h]) for h in range(4)]
                        for h in range(4):
                            for g in range(16):
                                off_ = 256 * h + 16 * g
                                plsc.addupdate(
                                    acc.at[pl.ds(off_, 16)],
                                    pvs[h] * kvbuf[jj, pl.ds(HO + off_, 16)])
                        return tuple(ps[h] + pvs[h] for h in range(4))

                    news = list(news)
                    return tuple(newm + news) + (wb,)

                fin = lax.fori_loop(0, nch, chunk_body,
                                    (minf,) * 4 + (zero16,) * 4 + (win_base0,))
                invs = [1.0 / (fin[4 + h] + 1e-16) for h in range(4)]

                @pl.when(issued0 == 1)
                def _():
                    pltpu.make_async_copy(
                        orow, out_hbm.at[pl.ds(0, OUT)], semo).wait()

                for g in range(16):
                    o = zero16
                    for h in range(4):
                        o = o + acc[pl.ds(256 * h + 16 * g, 16)] * invs[h]
                    orow[pl.ds(16 * g, 16)] = o * 0.25
                pltpu.async_copy(
                    orow, out_hbm.at[pl.ds((t * NPAD + d) * OUT, OUT)], semo)
                return (fin[8], jnp.int32(1))

            return lax.fori_loop(0, nloc, node_body, gcarry)

        gfin = lax.fori_loop(0, lax.div(ndst + 15, 16), grp_body,
                             (jnp.int32(-(1 << 30)), jnp.int32(0)))

        @pl.when(gfin[1] == 1)
        def _():
            pltpu.make_async_copy(
                orow, out_hbm.at[pl.ds(0, OUT)], semo).wait()


def _sc_conv(rp2, ssrcp, q0, kv0, q1, kv1):
    mesh = plsc.VectorSubcoreMesh(core_axis_name="c", subcore_axis_name="s")
    kfn = pl.kernel(
        _conv_body,
        out_type=jax.ShapeDtypeStruct((2 * NPAD * OUT,), F32),
        mesh=mesh,
        scratch_types=[
            pltpu.VMEM((672,), jnp.int32),
            pltpu.VMEM((264,), jnp.int32),
            pltpu.VMEM((16,), jnp.int32),
            pltpu.VMEM((16, HO), F32),
            pltpu.VMEM((16, 2 * HO), F32),
            pltpu.VMEM((HO,), F32),
            pltpu.VMEM((OUT,), F32),
            pltpu.VMEM((256,), F32),
            pltpu.VMEM((16, 64), F32),
            pltpu.SemaphoreType.DMA,
            pltpu.SemaphoreType.DMA,
        ],
    )
    out = kfn(rp2, ssrcp, q0, kv0, q1, kv1)
    return out.reshape(2, NPAD, OUT)


def kernel(cat_features, prop_features, tweet_features, des_features,
           edge_index, edge_type, params):
    src = edge_index[0].astype(jnp.int32)
    dst = edge_index[1].astype(jnp.int32)
    et = edge_type.astype(jnp.int32)
    key = et * N + dst
    order = jnp.argsort(key)
    skey = jnp.take(key, order)
    ssrc = jnp.take(src, order)
    rp = jnp.searchsorted(skey, jnp.arange(2 * N + 1, dtype=jnp.int32))
    rp = rp.astype(jnp.int32)
    rp2 = jnp.stack([rp[:N + 1], rp[N:2 * N + 1]])
    rp2 = jnp.pad(rp2, ((0, 0), (0, RPW - (N + 1))), mode="edge").reshape(-1)
    ssrcp = jnp.pad(ssrc, (0, 272))

    x = _encode(prop_features, cat_features, tweet_features, des_features,
                params)
    sem0 = sem1 = coef = None
    for li, rgt in enumerate((params["rgt1"], params["rgt2"])):
        if li == 0:
            q0, kv0, s0, q1, kv1, s1 = _qkvs(x, rgt)
        else:
            x, q0, kv0, s0, q1, kv1, s1 = _qkvs_from_sems(
                sem0, sem1, coef, rgt)
        conv = _sc_conv(rp2, ssrcp, q0, kv0, q1, kv1)
        sem0, sem1, coef = _gate(x, conv[0], conv[1], s0, s1, rgt)
    y = _final(sem0, sem1, coef, params)
    return y[:N, :2]
